# Initial kernel scaffold; baseline (speedup 1.0000x reference)
import jax
import jax.numpy as jnp
from jax.experimental import pallas as pl


def kernel(input, W_sweight, W_smu, W_svar, trans_mat_weight, trans_mat_mu, trans_mat_var):
    raise NotImplementedError("write your pallas kernel here")


# speedup vs baseline: 6.0689x; 6.0689x over previous
import jax
import jax.numpy as jnp
from jax import lax
from jax.experimental import pallas as pl
from jax.experimental.pallas import tpu as pltpu
from jax.experimental.pallas import tpu_sc as plsc

V = 100000
C = 64
K = 8
B = 1024
L = 200

NC, NS, LANES = 2, 16, 16
NW = NC * NS
NBS = B // 128
NLG = NW // NBS
LPW = L // NLG
LC = 2
NCHUNK = LPW // LC
TPC = LC * 128
NG = TPC // LANES

_SORT8 = [(0, 1), (2, 3), (4, 5), (6, 7), (0, 2), (1, 3), (4, 6), (5, 7),
          (1, 2), (5, 6), (0, 4), (3, 7), (1, 5), (2, 6), (1, 4), (3, 6),
          (2, 4), (3, 5), (3, 4)]
_BITONIC8 = [(0, 4), (1, 5), (2, 6), (3, 7), (0, 2), (1, 3), (4, 6), (5, 7),
             (0, 1), (2, 3), (4, 5), (6, 7)]


def _ce(v, ix, a, b):
    m = v[a] >= v[b]
    va = jnp.where(m, v[a], v[b])
    vb = jnp.where(m, v[b], v[a])
    ia = jnp.where(m, ix[a], ix[b])
    ib = jnp.where(m, ix[b], ix[a])
    v[a], v[b], ix[a], ix[b] = va, vb, ia, ib


def _merge_top8(A, B_):
    av, ai = A
    bv, bi = B_
    cv, ci = [], []
    for i in range(8):
        m = av[i] >= bv[7 - i]
        cv.append(jnp.where(m, av[i], bv[7 - i]))
        ci.append(jnp.where(m, ai[i], bi[7 - i]))
    for a, b in _BITONIC8:
        _ce(cv, ci, a, b)
    return cv, ci


def _top8(vals):
    groups = []
    for g in range(8):
        gv = [vals[8 * g + i] for i in range(8)]
        gi = [jnp.full((LANES,), 8 * g + i, jnp.int32) for i in range(8)]
        for a, b in _SORT8:
            _ce(gv, gi, a, b)
        groups.append((gv, gi))
    while len(groups) > 1:
        groups = [_merge_top8(groups[i], groups[i + 1])
                  for i in range(0, len(groups), 2)]
    return groups[0]


def _sc_body(ids_hbm, wsw_hbm, wmu_hbm, wvar_hbm, tvar_hbm,
             score_out, mu_out, var_out, tvar_out,
             ids_v, sw_v, mu_v, var_v, score_st, mu_st, var_st,
             tv_v, tv_st, sem):
    w = lax.axis_index("s") * NC + lax.axis_index("c")
    boff = (w % NBS) * 128
    lbase = (w // NBS) * LPW
    lane = jnp.arange(LANES, dtype=jnp.int32)

    pltpu.sync_copy(tvar_hbm.at[w], tv_v)
    for i in range(8):
        sl = pl.ds(i * 16, 16)
        a = tv_v[0, sl]
        bb = tv_v[1, sl]
        cc = tv_v[2, sl]
        o01 = bb * cc
        tv_st[0, sl] = a * a + bb * bb
        tv_st[1, sl] = o01
        tv_st[2, sl] = o01
        tv_st[3, sl] = cc * cc
    pltpu.sync_copy(tv_st, tvar_out.at[:, pl.ds(w * 128, 128)])

    @pl.loop(0, NCHUNK)
    def _chunk(ci):
        l0 = lbase + ci * LC
        for j in range(LC):
            pltpu.sync_copy(ids_hbm.at[pl.ds((l0 + j) * B + boff, 128)],
                            ids_v.at[pl.ds(j * 128, 128)])
        copies = []
        for j in range(LC):
            ids_j = ids_v.at[pl.ds(j * 128, 128)]
            dst = pl.ds(j * 128, 128)
            copies.append(pltpu.async_copy(wsw_hbm.at[ids_j],
                                           sw_v.at[dst], sem))
            copies.append(pltpu.async_copy(wmu_hbm.at[ids_j],
                                           mu_v.at[dst], sem))
            copies.append(pltpu.async_copy(wvar_hbm.at[ids_j],
                                           var_v.at[dst], sem))
        for cp in copies:
            cp.wait()

        @pl.loop(0, NG)
        def _group(g):
            j = g // 8
            off = (g % 8) * 16
            tidx = g * 16 + lane
            vals = [plsc.load_gather(sw_v, [tidx,
                                            jnp.full((LANES,), c, jnp.int32)])
                    for c in range(C)]
            tv, ti = _top8(vals)
            for k in range(K):
                score_st[j, k, pl.ds(off, 16)] = tv[k]
                mk = plsc.load_gather(mu_v, [tidx, ti[k]])
                mu_st[j, k, pl.ds(off, 16)] = mk
                vk = plsc.load_gather(var_v, [tidx, ti[k]])
                var_st[j, k, pl.ds(off, 16)] = vk * vk

        pltpu.sync_copy(score_st,
                        score_out.at[pl.ds(l0, LC), :, pl.ds(boff, 128)])
        pltpu.sync_copy(mu_st, mu_out.at[pl.ds(l0, LC), :, pl.ds(boff, 128)])
        pltpu.sync_copy(var_st, var_out.at[pl.ds(l0, LC), :, pl.ds(boff, 128)])


@jax.jit
def _sc_call(ids_flat, wsw, wmu, wvar, tvar_w):
    mesh = plsc.VectorSubcoreMesh(core_axis_name="c", subcore_axis_name="s",
                                  num_cores=NC, num_subcores=NS)
    kern = pl.kernel(
        _sc_body,
        out_type=(
            jax.ShapeDtypeStruct((L, K, B), jnp.float32),
            jax.ShapeDtypeStruct((L, K, B), jnp.float32),
            jax.ShapeDtypeStruct((L, K, B), jnp.float32),
            jax.ShapeDtypeStruct((4, C * C), jnp.float32),
        ),
        mesh=mesh,
        compiler_params=pltpu.CompilerParams(needs_layout_passes=False,
                                             use_tc_tiling_on_sc=False),
        scratch_types=[
            pltpu.VMEM((TPC,), jnp.int32),
            pltpu.VMEM((TPC, C), jnp.float32),
            pltpu.VMEM((TPC, C), jnp.float32),
            pltpu.VMEM((TPC, C), jnp.float32),
            pltpu.VMEM((LC, K, 128), jnp.float32),
            pltpu.VMEM((LC, K, 128), jnp.float32),
            pltpu.VMEM((LC, K, 128), jnp.float32),
            pltpu.VMEM((3, 128), jnp.float32),
            pltpu.VMEM((4, 128), jnp.float32),
            pltpu.SemaphoreType.DMA,
        ],
    )
    return kern(ids_flat, wsw, wmu, wvar, tvar_w)


def kernel(input, W_sweight, W_smu, W_svar, trans_mat_weight, trans_mat_mu,
           trans_mat_var):
    ids_flat = jnp.transpose(input).astype(jnp.int32).reshape(-1)
    tvar_w = (trans_mat_var.reshape(C * C, 3).T
              .reshape(3, NW, 128).transpose(1, 0, 2))
    score, mu, var, tvar_pl = _sc_call(ids_flat, W_sweight, W_smu, W_svar,
                                       tvar_w)
    t_weight = trans_mat_weight.reshape(1, C, C, 1)
    t_mu = trans_mat_mu.reshape(1, C, C, 2)
    t_var = tvar_pl.T.reshape(1, C, C, 2, 2)
    return (score, mu, var, t_weight, t_mu, t_var)


# speedup vs baseline: 7.5718x; 1.2476x over previous
import jax
import jax.numpy as jnp
from jax import lax
from jax.experimental import pallas as pl
from jax.experimental.pallas import tpu as pltpu
from jax.experimental.pallas import tpu_sc as plsc

V = 100000
C = 64
K = 8
B = 1024
L = 200

NC, NS, LANES = 2, 16, 16
NW = NC * NS
NBS = B // 128
LPW = L // (NW // NBS)
LC = 2
NCHUNK = LPW // LC
TPC = LC * 128
NG = TPC // LANES

_SORT8 = [(0, 1), (2, 3), (4, 5), (6, 7), (0, 2), (1, 3), (4, 6), (5, 7),
          (1, 2), (5, 6), (0, 4), (3, 7), (1, 5), (2, 6), (1, 4), (3, 6),
          (2, 4), (3, 5), (3, 4)]
_BITONIC8 = [(0, 4), (1, 5), (2, 6), (3, 7), (0, 2), (1, 3), (4, 6), (5, 7),
             (0, 1), (2, 3), (4, 5), (6, 7)]


def _ce(v, a, b):
    hi = jnp.maximum(v[a], v[b])
    lo = jnp.minimum(v[a], v[b])
    v[a], v[b] = hi, lo


def _top8_keys(keys):
    groups = []
    for g in range(8):
        gv = [keys[8 * g + i] for i in range(8)]
        for a, b in _SORT8:
            _ce(gv, a, b)
        groups.append(gv)
    while len(groups) > 1:
        merged = []
        for i in range(0, len(groups), 2):
            av, bv = groups[i], groups[i + 1]
            cv = [jnp.maximum(av[j], bv[7 - j]) for j in range(8)]
            for a, b in _BITONIC8:
                _ce(cv, a, b)
            merged.append(cv)
        groups = merged
    return groups[0]


def _sc_body(ids_hbm, wsw_hbm, wmu_hbm, wvar_hbm, tvar_hbm,
             score_out, mu_out, var_out, tvar_out,
             ids_v, sw0, mu0, var0, sw1, mu1, var1,
             score_st, mu_st, var_st, tv_v, tv_st, sem):
    w = lax.axis_index("s") * NC + lax.axis_index("c")
    boff = (w % NBS) * 128
    lbase = (w // NBS) * LPW
    lane = jnp.arange(LANES, dtype=jnp.int32)
    bufs = ((sw0, mu0, var0), (sw1, mu1, var1))
    tables = (wsw_hbm, wmu_hbm, wvar_hbm)

    pltpu.sync_copy(tvar_hbm.at[w], tv_v)
    for i in range(8):
        sl = pl.ds(i * 16, 16)
        a = tv_v[0, sl]
        bb = tv_v[1, sl]
        cc = tv_v[2, sl]
        o01 = bb * cc
        tv_st[0, sl] = a * a + bb * bb
        tv_st[1, sl] = o01
        tv_st[2, sl] = o01
        tv_st[3, sl] = cc * cc
    pltpu.sync_copy(tv_st, tvar_out.at[:, pl.ds(w * 128, 128)])

    def load_ids(ci, slot):
        l0 = lbase + ci * LC
        for j in range(LC):
            pltpu.sync_copy(ids_hbm.at[pl.ds((l0 + j) * B + boff, 128)],
                            ids_v.at[slot, pl.ds(j * 128, 128)])

    def fire(slot):
        for j in range(LC):
            idx = ids_v.at[slot, pl.ds(j * 128, 128)]
            d = pl.ds(j * 128, 128)
            for t in range(3):
                pltpu.async_copy(tables[t].at[idx], bufs[slot][t].at[d], sem)

    def drain(slot):
        for t in range(3):
            pltpu.make_async_copy(tables[t].at[pl.ds(0, TPC)],
                                  bufs[slot][t], sem).wait()

    def compute(ci, slot):
        swb, mub, varb = bufs[slot]

        @pl.loop(0, NG)
        def _group(g):
            j = g // 8
            off = (g % 8) * 16
            tidx = g * 16 + lane
            keys = []
            for c in range(C):
                v = plsc.load_gather(swb, [tidx,
                                           jnp.full((LANES,), c, jnp.int32)])
                u = plsc.bitcast(v, jnp.int32)
                keys.append(plsc.bitcast((u | 63) - c, jnp.float32))
            kv = _top8_keys(keys)
            for k in range(K):
                ki = plsc.bitcast(kv[k], jnp.int32)
                idx = 63 - (ki & 63)
                sl = pl.ds(off, 16)
                score_st[j, k, sl] = plsc.load_gather(swb, [tidx, idx])
                mu_st[j, k, sl] = plsc.load_gather(mub, [tidx, idx])
                vv = plsc.load_gather(varb, [tidx, idx])
                var_st[j, k, sl] = vv * vv

        l0 = lbase + ci * LC
        pltpu.sync_copy(score_st,
                        score_out.at[pl.ds(l0, LC), :, pl.ds(boff, 128)])
        pltpu.sync_copy(mu_st, mu_out.at[pl.ds(l0, LC), :, pl.ds(boff, 128)])
        pltpu.sync_copy(var_st, var_out.at[pl.ds(l0, LC), :, pl.ds(boff, 128)])

    load_ids(0, 0)
    fire(0)

    @pl.loop(0, (NCHUNK - 1) // 2)
    def _iter(it):
        ci0 = 2 * it
        load_ids(ci0 + 1, 1)
        fire(1)
        drain(0)
        compute(ci0, 0)
        load_ids(ci0 + 2, 0)
        fire(0)
        drain(1)
        compute(ci0 + 1, 1)

    drain(0)
    compute(NCHUNK - 1, 0)


@jax.jit
def _sc_call(ids_flat, wsw, wmu, wvar, tvar_w):
    mesh = plsc.VectorSubcoreMesh(core_axis_name="c", subcore_axis_name="s",
                                  num_cores=NC, num_subcores=NS)
    kern = pl.kernel(
        _sc_body,
        out_type=(
            jax.ShapeDtypeStruct((L, K, B), jnp.float32),
            jax.ShapeDtypeStruct((L, K, B), jnp.float32),
            jax.ShapeDtypeStruct((L, K, B), jnp.float32),
            jax.ShapeDtypeStruct((4, C * C), jnp.float32),
        ),
        mesh=mesh,
        compiler_params=pltpu.CompilerParams(needs_layout_passes=False,
                                             use_tc_tiling_on_sc=False),
        scratch_types=[
            pltpu.VMEM((2, TPC), jnp.int32),
            pltpu.VMEM((TPC, C), jnp.float32),
            pltpu.VMEM((TPC, C), jnp.float32),
            pltpu.VMEM((TPC, C), jnp.float32),
            pltpu.VMEM((TPC, C), jnp.float32),
            pltpu.VMEM((TPC, C), jnp.float32),
            pltpu.VMEM((TPC, C), jnp.float32),
            pltpu.VMEM((LC, K, 128), jnp.float32),
            pltpu.VMEM((LC, K, 128), jnp.float32),
            pltpu.VMEM((LC, K, 128), jnp.float32),
            pltpu.VMEM((3, 128), jnp.float32),
            pltpu.VMEM((4, 128), jnp.float32),
            pltpu.SemaphoreType.DMA,
        ],
    )
    return kern(ids_flat, wsw, wmu, wvar, tvar_w)


def kernel(input, W_sweight, W_smu, W_svar, trans_mat_weight, trans_mat_mu,
           trans_mat_var):
    ids_flat = jnp.transpose(input).astype(jnp.int32).reshape(-1)
    tvar_w = (trans_mat_var.reshape(C * C, 3).T
              .reshape(3, NW, 128).transpose(1, 0, 2))
    score, mu, var, tvar_pl = _sc_call(ids_flat, W_sweight, W_smu, W_svar,
                                       tvar_w)
    t_weight = trans_mat_weight.reshape(1, C, C, 1)
    t_mu = trans_mat_mu.reshape(1, C, C, 2)
    t_var = tvar_pl.T.reshape(1, C, C, 2, 2)
    return (score, mu, var, t_weight, t_mu, t_var)


# speedup vs baseline: 11.2305x; 1.4832x over previous
import jax
import jax.numpy as jnp
from jax import lax
from jax.experimental import pallas as pl
from jax.experimental.pallas import tpu as pltpu
from jax.experimental.pallas import tpu_sc as plsc

V = 100000
C = 64
K = 8
B = 1024
L = 200

NC, NS, LANES = 2, 16, 16
NW = NC * NS
NBS = B // 128
NLG = NW // NBS
LPW = L // NLG
LC = 2
NCHUNK = LPW // LC
TPC = LC * 128
NG = TPC // LANES

_SORT8 = [(0, 1), (2, 3), (4, 5), (6, 7), (0, 2), (1, 3), (4, 6), (5, 7),
          (1, 2), (5, 6), (0, 4), (3, 7), (1, 5), (2, 6), (1, 4), (3, 6),
          (2, 4), (3, 5), (3, 4)]
_BITONIC8 = [(0, 4), (1, 5), (2, 6), (3, 7), (0, 2), (1, 3), (4, 6), (5, 7),
             (0, 1), (2, 3), (4, 5), (6, 7)]


def _ce(v, a, b):
    hi = jnp.maximum(v[a], v[b])
    lo = jnp.minimum(v[a], v[b])
    v[a], v[b] = hi, lo


def _top8_keys(keys):
    groups = []
    for g in range(8):
        gv = [keys[8 * g + i] for i in range(8)]
        for a, b in _SORT8:
            _ce(gv, a, b)
        groups.append(gv)
    while len(groups) > 1:
        merged = []
        for i in range(0, len(groups), 2):
            av, bv = groups[i], groups[i + 1]
            cv = [jnp.maximum(av[j], bv[7 - j]) for j in range(8)]
            for a, b in _BITONIC8:
                _ce(cv, a, b)
            merged.append(cv)
        groups = merged
    return groups[0]


def _sc_body(ids_hbm, wsw_hbm, wmu_hbm, wvar_hbm, tvar_hbm,
             score_out, mu_out, var_out, tvar_out,
             ids_all, sw0, mu0, var0, sw1, mu1, var1,
             score_st, mu_st, var_st, tv_v, tv_st, sem):
    w = lax.axis_index("s") * NC + lax.axis_index("c")
    sb = w % NBS
    boff = sb * 128
    lg = w // NBS
    lbase = lg * LPW
    lane = jnp.arange(LANES, dtype=jnp.int32)
    bufs = ((sw0, mu0, var0), (sw1, mu1, var1))
    tables = (wsw_hbm, wmu_hbm, wvar_hbm)

    pltpu.sync_copy(ids_hbm.at[lg, :, pl.ds(boff, 128)], ids_all)

    pltpu.sync_copy(tvar_hbm.at[w], tv_v)
    for i in range(8):
        sl = pl.ds(i * 16, 16)
        a = tv_v[0, sl]
        bb = tv_v[1, sl]
        cc = tv_v[2, sl]
        o01 = bb * cc
        tv_st[0, sl] = a * a + bb * bb
        tv_st[1, sl] = o01
        tv_st[2, sl] = o01
        tv_st[3, sl] = cc * cc
    pltpu.sync_copy(tv_st, tvar_out.at[:, pl.ds(w * 128, 128)])

    def fire(ci, slot):
        for j in range(LC):
            idx = ids_all.at[ci * LC + j]
            d = pl.ds(j * 128, 128)
            for t in range(3):
                pltpu.async_copy(tables[t].at[idx], bufs[slot][t].at[d], sem)

    def drain(slot):
        for t in range(3):
            pltpu.make_async_copy(tables[t].at[pl.ds(0, TPC)],
                                  bufs[slot][t], sem).wait()

    def compute(ci, slot):
        swb, mub, varb = bufs[slot]

        @pl.loop(0, NG)
        def _group(g):
            j = g // 8
            off = (g % 8) * 16
            tidx = g * 16 + lane
            keys = []
            for i in range(C):
                col = (lane + i) & 63
                v = plsc.load_gather(swb, [tidx, col])
                u = plsc.bitcast(v, jnp.int32)
                keys.append(plsc.bitcast((u | 63) - col, jnp.float32))
            kv = _top8_keys(keys)
            for k in range(K):
                ki = plsc.bitcast(kv[k], jnp.int32)
                idx = 63 - (ki & 63)
                sl = pl.ds(off, 16)
                score_st[j, k, sl] = plsc.load_gather(swb, [tidx, idx])
                mu_st[j, k, sl] = plsc.load_gather(mub, [tidx, idx])
                vv = plsc.load_gather(varb, [tidx, idx])
                var_st[j, k, sl] = vv * vv

        l0 = lbase + ci * LC
        pltpu.sync_copy(score_st, score_out.at[pl.ds(l0, LC), sb])
        pltpu.sync_copy(mu_st, mu_out.at[pl.ds(l0, LC), sb])
        pltpu.sync_copy(var_st, var_out.at[pl.ds(l0, LC), sb])

    fire(0, 0)

    @pl.loop(0, (NCHUNK - 1) // 2)
    def _iter(it):
        ci0 = 2 * it
        fire(ci0 + 1, 1)
        drain(0)
        compute(ci0, 0)
        fire(ci0 + 2, 0)
        drain(1)
        compute(ci0 + 1, 1)

    drain(0)
    compute(NCHUNK - 1, 0)


@jax.jit
def _sc_call(ids_4d, wsw, wmu, wvar, tvar_w):
    mesh = plsc.VectorSubcoreMesh(core_axis_name="c", subcore_axis_name="s",
                                  num_cores=NC, num_subcores=NS)
    kern = pl.kernel(
        _sc_body,
        out_type=(
            jax.ShapeDtypeStruct((L, NBS, K, 128), jnp.float32),
            jax.ShapeDtypeStruct((L, NBS, K, 128), jnp.float32),
            jax.ShapeDtypeStruct((L, NBS, K, 128), jnp.float32),
            jax.ShapeDtypeStruct((4, C * C), jnp.float32),
        ),
        mesh=mesh,
        compiler_params=pltpu.CompilerParams(needs_layout_passes=False,
                                             use_tc_tiling_on_sc=False),
        scratch_types=[
            pltpu.VMEM((LPW, 128), jnp.int32),
            pltpu.VMEM((TPC, C), jnp.float32),
            pltpu.VMEM((TPC, C), jnp.float32),
            pltpu.VMEM((TPC, C), jnp.float32),
            pltpu.VMEM((TPC, C), jnp.float32),
            pltpu.VMEM((TPC, C), jnp.float32),
            pltpu.VMEM((TPC, C), jnp.float32),
            pltpu.VMEM((LC, K, 128), jnp.float32),
            pltpu.VMEM((LC, K, 128), jnp.float32),
            pltpu.VMEM((LC, K, 128), jnp.float32),
            pltpu.VMEM((3, 128), jnp.float32),
            pltpu.VMEM((4, 128), jnp.float32),
            pltpu.SemaphoreType.DMA,
        ],
    )
    return kern(ids_4d, wsw, wmu, wvar, tvar_w)


def kernel(input, W_sweight, W_smu, W_svar, trans_mat_weight, trans_mat_mu,
           trans_mat_var):
    ids_4d = jnp.transpose(input).astype(jnp.int32).reshape(NLG, LPW, B)
    tvar_w = (trans_mat_var.reshape(C * C, 3).T
              .reshape(3, NW, 128).transpose(1, 0, 2))
    score4, mu4, var4, tvar_pl = _sc_call(ids_4d, W_sweight, W_smu, W_svar,
                                          tvar_w)
    score = score4.transpose(0, 2, 1, 3).reshape(L, K, B)
    mu = mu4.transpose(0, 2, 1, 3).reshape(L, K, B)
    var = var4.transpose(0, 2, 1, 3).reshape(L, K, B)
    t_weight = trans_mat_weight.reshape(1, C, C, 1)
    t_mu = trans_mat_mu.reshape(1, C, C, 2)
    t_var = tvar_pl.T.reshape(1, C, C, 2, 2)
    return (score, mu, var, t_weight, t_mu, t_var)
